# Initial kernel scaffold; baseline (speedup 1.0000x reference)
import jax
import jax.numpy as jnp
from jax.experimental import pallas as pl


def kernel(state, R_s, R_r, dt, eW1, eb1, eW2, eb2, nW1, nb1, nW2, nb2, nW3, nb3, gW1, gb1, gW2, gb2, lW, lb):
    raise NotImplementedError("write your pallas kernel here")


# speedup vs baseline: 14.8063x; 14.8063x over previous
import functools

import jax
import jax.numpy as jnp
from jax import lax
from jax.experimental import pallas as pl
from jax.experimental.pallas import tpu as pltpu
from jax.experimental.pallas import tpu_sc as plsc

BOX = 6.0
NC, NS = 2, 16
NW = NC * NS
CH = 128
F32 = jnp.float32


def _sp(x):
    return jnp.maximum(x, 0.0) + jnp.log1p(jnp.exp(-jnp.abs(x)))


def _sg(x):
    return 1.0 / (1.0 + jnp.exp(-x))


def _mm(a, b):
    return jnp.dot(a, b, preferred_element_type=F32)


def _sc_mesh():
    return plsc.VectorSubcoreMesh(core_axis_name="c", subcore_axis_name="s")


_SC_PARAMS = pltpu.CompilerParams(use_tc_tiling_on_sc=False)


def _wid():
    return lax.axis_index("s") * NC + lax.axis_index("c")


def _make_gather2(E, D):
    nchunks = E // CH

    @functools.partial(
        pl.kernel,
        out_type=[jax.ShapeDtypeStruct((E, D), F32),
                  jax.ShapeDtypeStruct((E, D), F32)],
        mesh=_sc_mesh(),
        compiler_params=_SC_PARAMS,
        scratch_types=[pltpu.VMEM((CH,), jnp.int32),
                       pltpu.VMEM((CH, D), F32),
                       pltpu.SemaphoreType.DMA],
    )
    def k(t_hbm, rs_hbm, rr_hbm, s_out, r_out, idx_v, rows_v, sem):
        w = _wid()
        tc = (nchunks - w + NW - 1) // NW

        def body(j, _):
            c = w + j * NW
            base = c * CH
            pltpu.sync_copy(rs_hbm.at[c], idx_v)
            pltpu.async_copy(t_hbm.at[idx_v], rows_v, sem).wait()
            pltpu.sync_copy(rows_v, s_out.at[pl.ds(base, CH)])
            pltpu.sync_copy(rr_hbm.at[c], idx_v)
            pltpu.async_copy(t_hbm.at[idx_v], rows_v, sem).wait()
            pltpu.sync_copy(rows_v, r_out.at[pl.ds(base, CH)])
            return _

        lax.fori_loop(0, tc, body, None)

    return k


def _make_gather1(E, D):
    nchunks = E // CH

    @functools.partial(
        pl.kernel,
        out_type=jax.ShapeDtypeStruct((E, D), F32),
        mesh=_sc_mesh(),
        compiler_params=_SC_PARAMS,
        scratch_types=[pltpu.VMEM((CH,), jnp.int32),
                       pltpu.VMEM((CH, D), F32),
                       pltpu.SemaphoreType.DMA],
    )
    def k(t_hbm, rr_hbm, out, idx_v, rows_v, sem):
        w = _wid()
        tc = (nchunks - w + NW - 1) // NW

        def body(j, _):
            c = w + j * NW
            base = c * CH
            pltpu.sync_copy(rr_hbm.at[c], idx_v)
            pltpu.async_copy(t_hbm.at[idx_v], rows_v, sem).wait()
            pltpu.sync_copy(rows_v, out.at[pl.ds(base, CH)])
            return _

        lax.fori_loop(0, tc, body, None)

    return k


def _make_scatter1(E, Np, D):
    nchunks = E // CH
    rows_per_sub = Np // NS

    @functools.partial(
        pl.kernel,
        out_type=jax.ShapeDtypeStruct((NC, Np, D), F32),
        mesh=_sc_mesh(),
        compiler_params=_SC_PARAMS,
        scratch_types=[pltpu.VMEM((CH,), jnp.int32),
                       pltpu.VMEM((CH, D), F32),
                       pltpu.VMEM_SHARED((Np, D), F32)],
    )
    def k(pay_hbm, idx_hbm, zero_hbm, out, idx_v, pay_v, table):
        w = _wid()
        cid = lax.axis_index("c")
        sid = lax.axis_index("s")
        sl = pl.ds(sid * rows_per_sub, rows_per_sub)
        pltpu.sync_copy(zero_hbm.at[sl], table.at[sl])
        plsc.subcore_barrier()
        tc = (nchunks - w + NW - 1) // NW

        def body(j, _):
            c = w + j * NW
            base = c * CH
            pltpu.sync_copy(idx_hbm.at[c], idx_v)
            pltpu.sync_copy(pay_hbm.at[pl.ds(base, CH)], pay_v)
            pltpu.sync_copy(pay_v, table.at[idx_v], add=True)
            return _

        lax.fori_loop(0, tc, body, None)
        plsc.subcore_barrier()
        pltpu.sync_copy(table.at[sl], out.at[cid, sl])

    return k


def _make_scatter2(E, Np, D):
    nchunks = E // CH
    rows_per_sub = Np // NS

    @functools.partial(
        pl.kernel,
        out_type=[jax.ShapeDtypeStruct((NC, Np, D), F32),
                  jax.ShapeDtypeStruct((NC, Np, D), F32)],
        mesh=_sc_mesh(),
        compiler_params=_SC_PARAMS,
        scratch_types=[pltpu.VMEM((CH,), jnp.int32),
                       pltpu.VMEM((CH, D), F32),
                       pltpu.VMEM_SHARED((Np, D), F32),
                       pltpu.VMEM_SHARED((Np, D), F32)],
    )
    def k(ps_hbm, pr_hbm, rs_hbm, rr_hbm, zero_hbm, out_s, out_r,
          idx_v, pay_v, tab_s, tab_r):
        w = _wid()
        cid = lax.axis_index("c")
        sid = lax.axis_index("s")
        sl = pl.ds(sid * rows_per_sub, rows_per_sub)
        pltpu.sync_copy(zero_hbm.at[sl], tab_s.at[sl])
        pltpu.sync_copy(zero_hbm.at[sl], tab_r.at[sl])
        plsc.subcore_barrier()
        tc = (nchunks - w + NW - 1) // NW

        def body(j, _):
            c = w + j * NW
            base = c * CH
            pltpu.sync_copy(rs_hbm.at[c], idx_v)
            pltpu.sync_copy(ps_hbm.at[pl.ds(base, CH)], pay_v)
            pltpu.sync_copy(pay_v, tab_s.at[idx_v], add=True)
            pltpu.sync_copy(rr_hbm.at[c], idx_v)
            pltpu.sync_copy(pr_hbm.at[pl.ds(base, CH)], pay_v)
            pltpu.sync_copy(pay_v, tab_r.at[idx_v], add=True)
            return _

        lax.fori_loop(0, tc, body, None)
        plsc.subcore_barrier()
        pltpu.sync_copy(tab_s.at[sl], out_s.at[cid, sl])
        pltpu.sync_copy(tab_r.at[sl], out_r.at[cid, sl])

    return k


def _edge_fwd_body(s_ref, r_ref, eW1_ref, eb1_ref, eW2_ref, eb2_ref,
                   nW1e_ref, m_ref, se_ref):
    S = s_ref[...]
    R = r_ref[...]
    dpos = S[:, 3:5] - R[:, 3:5]
    dpos = jnp.where(dpos > BOX / 2, dpos - BOX, dpos)
    dpos = jnp.where(dpos <= -BOX / 2, dpos + BOX, dpos)
    Ef = jnp.concatenate([S[:, 0:3], R[:, 0:3], dpos], axis=1)
    A1 = _sp(_mm(Ef, eW1_ref[...]) + eb1_ref[...])
    Eh = _sp(_mm(A1, eW2_ref[...]) + eb2_ref[...])
    M = _mm(Eh, nW1e_ref[...])
    m_ref[...] = jnp.concatenate(
        [M, jnp.zeros((M.shape[0], 12), F32)], axis=1)

    @pl.when(pl.program_id(0) == 0)
    def _():
        se_ref[...] = jnp.zeros_like(se_ref)

    se_ref[...] += jnp.sum(Eh, axis=0, keepdims=True)


def _node_global_body(n_real, t_ref, agg_ref, se_ref,
                      nW1n_ref, nb1_ref, nW2_ref, nb2_ref, nW3_ref, nb3_ref,
                      gW1_ref, gb1_ref, gW2_ref, gb2_ref, lT_ref,
                      nW1nT_ref, nW2T_ref, nW3T_ref, gW1T_ref, gW2T_ref,
                      g_ref, dvnp_ref, dse_ref):
    Vnp = t_ref[...][:, 0:3]
    agg2 = agg_ref[...]
    agg = agg2[0, :, 0:100] + agg2[1, :, 0:100]
    zN1 = _mm(Vnp, nW1n_ref[...]) + agg + nb1_ref[...]
    s1 = _sg(zN1)
    Nh1 = _sp(zN1)
    zN2 = _mm(Nh1, nW2_ref[...]) + nb2_ref[...]
    s2 = _sg(zN2)
    Nh2 = _sp(zN2)
    zN3 = _mm(Nh2, nW3_ref[...]) + nb3_ref[...]
    s3 = _sg(zN3)
    Nh = _sp(zN3)
    rows = lax.broadcasted_iota(jnp.int32, Nh.shape, 0)
    SN = jnp.sum(jnp.where(rows < n_real, Nh, 0.0), axis=0, keepdims=True)
    Gin = jnp.concatenate([se_ref[...], SN], axis=1)
    zG1 = _mm(Gin, gW1_ref[...]) + gb1_ref[...]
    Gh1 = _sp(zG1)
    zG2 = _mm(Gh1, gW2_ref[...]) + gb2_ref[...]
    dGh1 = _mm(lT_ref[...] * _sg(zG2), gW2T_ref[...])
    dGin = _mm(dGh1 * _sg(zG1), gW1T_ref[...])
    dSE = dGin[:, 0:150]
    dSN = dGin[:, 150:250]
    dNh2 = _mm(dSN * s3, nW3T_ref[...])
    dNh1 = _mm(dNh2 * s2, nW2T_ref[...])
    G = dNh1 * s1
    dvnp = _mm(G, nW1nT_ref[...])
    g_ref[...] = jnp.concatenate(
        [G, jnp.zeros((G.shape[0], 12), F32)], axis=1)
    dvnp_ref[...] = jnp.concatenate(
        [dvnp, jnp.zeros((dvnp.shape[0], 13), F32)], axis=1)
    dse_ref[...] = dSE


def _edge_bwd_body(s_ref, r_ref, grr_ref, dse_ref,
                   eW1_ref, eb1_ref, eW2_ref, eb2_ref,
                   nW1eT_ref, eW2T_ref, eW1T_ref,
                   ps_ref, pr_ref):
    S = s_ref[...]
    R = r_ref[...]
    dpos = S[:, 3:5] - R[:, 3:5]
    dpos = jnp.where(dpos > BOX / 2, dpos - BOX, dpos)
    dpos = jnp.where(dpos <= -BOX / 2, dpos + BOX, dpos)
    Ef = jnp.concatenate([S[:, 0:3], R[:, 0:3], dpos], axis=1)
    z1 = _mm(Ef, eW1_ref[...]) + eb1_ref[...]
    A1 = _sp(z1)
    z2 = _mm(A1, eW2_ref[...]) + eb2_ref[...]
    dEh = dse_ref[...] + _mm(grr_ref[...][:, 0:100], nW1eT_ref[...])
    dEh1 = _mm(dEh * _sg(z2), eW2T_ref[...])
    dEf = _mm(dEh1 * _sg(z1), eW1T_ref[...])
    zpad = jnp.zeros((dEf.shape[0], 11), F32)
    ps_ref[...] = jnp.concatenate(
        [dEf[:, 0:3], dEf[:, 6:8], zpad], axis=1)
    pr_ref[...] = jnp.concatenate(
        [dEf[:, 3:6], -dEf[:, 6:8], zpad], axis=1)


def _final_body(v_ref, dt_ref, ts_ref, tr_ref, dvnp_ref, out_ref):
    n = v_ref.shape[1]
    V = v_ref[0]
    ts2 = ts_ref[...]
    tr2 = tr_ref[...]
    ts = (ts2[0] + ts2[1])[:n]
    tr = (tr2[0] + tr2[1])[:n]
    dvnp_n = dvnp_ref[...][:n]
    dq = ts[:, 1:3] + tr[:, 1:3] + dvnp_n[:, 1:3]
    dvp = ts[:, 3:5] + tr[:, 3:5]
    dt = dt_ref[...]
    q = V[:, 1:3] + dt * dq
    q = jnp.where(q >= BOX / 2, q - BOX, q)
    q = jnp.where(q < -BOX / 2, q + BOX, q)
    p = V[:, 3:5] - dt * dvp
    out_ref[...] = jnp.concatenate([V[:, 0:1], q, p], axis=1)[None]


def _full(shape):
    return pl.BlockSpec(shape, lambda: tuple(0 for _ in shape))


def kernel(state, R_s, R_r, dt, eW1, eb1, eW2, eb2, nW1, nb1, nW2, nb2,
           nW3, nb3, gW1, gb1, gW2, gb2, lW, lb):
    B, N, _ = state.shape
    E = R_s.shape[1]
    Np = ((N + CH - 1) // CH) * CH
    EO = eW1.shape[1]
    NO = nW1.shape[1]
    BLK = 2000
    grid_e = E // BLK

    V = state[0]
    rs = R_s[0, :, 0].astype(jnp.int32)
    rr = R_r[0, :, 0].astype(jnp.int32)
    rs2 = rs.reshape(E // CH, CH)
    rr2 = rr.reshape(E // CH, CH)

    T16 = jnp.concatenate(
        [V[:, 0:1], V[:, 3:5], V[:, 1:3], jnp.zeros((N, 11), F32)], axis=1)
    T16 = jnp.concatenate([T16, jnp.zeros((Np - N, 16), F32)], axis=0)

    nW1n, nW1e = nW1[:3], nW1[3:]
    zeros112 = jnp.zeros((Np, 112), F32)
    zeros16 = jnp.zeros((Np, 16), F32)

    S16, R16 = _make_gather2(E, 16)(T16, rs2, rr2)

    M112, SE = pl.pallas_call(
        _edge_fwd_body,
        grid=(grid_e,),
        in_specs=[
            pl.BlockSpec((BLK, 16), lambda i: (i, 0)),
            pl.BlockSpec((BLK, 16), lambda i: (i, 0)),
            pl.BlockSpec((8, EO), lambda i: (0, 0)),
            pl.BlockSpec((1, EO), lambda i: (0, 0)),
            pl.BlockSpec((EO, EO), lambda i: (0, 0)),
            pl.BlockSpec((1, EO), lambda i: (0, 0)),
            pl.BlockSpec((EO, NO), lambda i: (0, 0)),
        ],
        out_specs=[
            pl.BlockSpec((BLK, 112), lambda i: (i, 0)),
            pl.BlockSpec((1, EO), lambda i: (0, 0)),
        ],
        out_shape=[
            jax.ShapeDtypeStruct((E, 112), F32),
            jax.ShapeDtypeStruct((1, EO), F32),
        ],
    )(S16, R16, eW1, eb1.reshape(1, EO), eW2, eb2.reshape(1, EO), nW1e)

    aggM = _make_scatter1(E, Np, 112)(M112, rr2, zeros112)

    G112, dVnp16, dSE = pl.pallas_call(
        functools.partial(_node_global_body, N),
        in_specs=[
            _full((Np, 16)), _full((NC, Np, 112)), _full((1, EO)),
            _full((3, NO)), _full((1, NO)),
            _full((NO, NO)), _full((1, NO)),
            _full((NO, NO)), _full((1, NO)),
            _full((EO + NO, NO)), _full((1, NO)),
            _full((NO, NO)), _full((1, NO)),
            _full((1, NO)),
            _full((NO, 3)), _full((NO, NO)), _full((NO, NO)),
            _full((NO, EO + NO)), _full((NO, NO)),
        ],
        out_specs=[_full((Np, 112)), _full((Np, 16)), _full((1, EO))],
        out_shape=[
            jax.ShapeDtypeStruct((Np, 112), F32),
            jax.ShapeDtypeStruct((Np, 16), F32),
            jax.ShapeDtypeStruct((1, EO), F32),
        ],
    )(T16, aggM, SE,
      nW1n, nb1.reshape(1, NO), nW2, nb2.reshape(1, NO), nW3,
      nb3.reshape(1, NO), gW1, gb1.reshape(1, NO), gW2, gb2.reshape(1, NO),
      lW.reshape(1, NO),
      nW1n.T, nW2.T, nW3.T, gW1.T, gW2.T)

    Grr = _make_gather1(E, 112)(G112, rr2)

    Prs, Prr = pl.pallas_call(
        _edge_bwd_body,
        grid=(grid_e,),
        in_specs=[
            pl.BlockSpec((BLK, 16), lambda i: (i, 0)),
            pl.BlockSpec((BLK, 16), lambda i: (i, 0)),
            pl.BlockSpec((BLK, 112), lambda i: (i, 0)),
            pl.BlockSpec((1, EO), lambda i: (0, 0)),
            pl.BlockSpec((8, EO), lambda i: (0, 0)),
            pl.BlockSpec((1, EO), lambda i: (0, 0)),
            pl.BlockSpec((EO, EO), lambda i: (0, 0)),
            pl.BlockSpec((1, EO), lambda i: (0, 0)),
            pl.BlockSpec((NO, EO), lambda i: (0, 0)),
            pl.BlockSpec((EO, EO), lambda i: (0, 0)),
            pl.BlockSpec((EO, 8), lambda i: (0, 0)),
        ],
        out_specs=[
            pl.BlockSpec((BLK, 16), lambda i: (i, 0)),
            pl.BlockSpec((BLK, 16), lambda i: (i, 0)),
        ],
        out_shape=[
            jax.ShapeDtypeStruct((E, 16), F32),
            jax.ShapeDtypeStruct((E, 16), F32),
        ],
    )(S16, R16, Grr, dSE, eW1, eb1.reshape(1, EO), eW2, eb2.reshape(1, EO),
      nW1e.T, eW2.T, eW1.T)

    TS, TR = _make_scatter2(E, Np, 16)(Prs, Prr, rs2, rr2, zeros16)

    out = pl.pallas_call(
        _final_body,
        in_specs=[
            _full((B, N, 5)),
            _full((N, 1)),
            _full((NC, Np, 16)), _full((NC, Np, 16)),
            _full((Np, 16)),
        ],
        out_specs=_full((B, N, 5)),
        out_shape=jax.ShapeDtypeStruct((B, N, 5), F32),
    )(state, dt[0].reshape(N, 1), TS, TR, dVnp16)

    return out


# speedup vs baseline: 16.0322x; 1.0828x over previous
import functools

import jax
import jax.numpy as jnp
from jax import lax
from jax.experimental import pallas as pl
from jax.experimental.pallas import tpu as pltpu
from jax.experimental.pallas import tpu_sc as plsc

BOX = 6.0
NC, NS = 2, 16
NW = NC * NS
CH = 128
F32 = jnp.float32


def _sp(x):
    return jnp.maximum(x, 0.0) + jnp.log1p(jnp.exp(-jnp.abs(x)))


def _sg(x):
    return 1.0 / (1.0 + jnp.exp(-x))


def _spsg(x):
    t = jnp.exp(-jnp.abs(x))
    sp = jnp.maximum(x, 0.0) + jnp.log1p(t)
    sg = 1.0 - jnp.exp(-sp)
    return sp, sg


def _mm(a, b):
    return jnp.dot(a, b, preferred_element_type=F32)


def _mmb(a, b):
    return jnp.dot(a.astype(jnp.bfloat16), b, preferred_element_type=F32)


def _sc_mesh():
    return plsc.VectorSubcoreMesh(core_axis_name="c", subcore_axis_name="s")


_SC_PARAMS = pltpu.CompilerParams(use_tc_tiling_on_sc=False)


def _wid():
    return lax.axis_index("s") * NC + lax.axis_index("c")


def _make_gather2(E, D):
    nchunks = E // CH

    @functools.partial(
        pl.kernel,
        out_type=[jax.ShapeDtypeStruct((E, D), F32),
                  jax.ShapeDtypeStruct((E, D), F32)],
        mesh=_sc_mesh(),
        compiler_params=_SC_PARAMS,
        scratch_types=[pltpu.VMEM((CH,), jnp.int32),
                       pltpu.VMEM((CH, D), F32),
                       pltpu.SemaphoreType.DMA],
    )
    def k(t_hbm, rs_hbm, rr_hbm, s_out, r_out, idx_v, rows_v, sem):
        w = _wid()
        tc = (nchunks - w + NW - 1) // NW

        def body(j, _):
            c = w + j * NW
            base = c * CH
            pltpu.sync_copy(rs_hbm.at[c], idx_v)
            pltpu.async_copy(t_hbm.at[idx_v], rows_v, sem).wait()
            pltpu.sync_copy(rows_v, s_out.at[pl.ds(base, CH)])
            pltpu.sync_copy(rr_hbm.at[c], idx_v)
            pltpu.async_copy(t_hbm.at[idx_v], rows_v, sem).wait()
            pltpu.sync_copy(rows_v, r_out.at[pl.ds(base, CH)])
            return _

        lax.fori_loop(0, tc, body, None)

    return k


def _make_gather1(E, D):
    nchunks = E // CH

    @functools.partial(
        pl.kernel,
        out_type=jax.ShapeDtypeStruct((E, D), F32),
        mesh=_sc_mesh(),
        compiler_params=_SC_PARAMS,
        scratch_types=[pltpu.VMEM((CH,), jnp.int32),
                       pltpu.VMEM((CH, D), F32),
                       pltpu.SemaphoreType.DMA],
    )
    def k(t_hbm, rr_hbm, out, idx_v, rows_v, sem):
        w = _wid()
        tc = (nchunks - w + NW - 1) // NW

        def body(j, _):
            c = w + j * NW
            base = c * CH
            pltpu.sync_copy(rr_hbm.at[c], idx_v)
            pltpu.async_copy(t_hbm.at[idx_v], rows_v, sem).wait()
            pltpu.sync_copy(rows_v, out.at[pl.ds(base, CH)])
            return _

        lax.fori_loop(0, tc, body, None)

    return k


def _make_scatter1(E, Np, D):
    nchunks = E // CH
    rows_per_sub = Np // NS

    @functools.partial(
        pl.kernel,
        out_type=jax.ShapeDtypeStruct((NC, Np, D), F32),
        mesh=_sc_mesh(),
        compiler_params=_SC_PARAMS,
        scratch_types=[pltpu.VMEM((CH,), jnp.int32),
                       pltpu.VMEM((CH, D), F32),
                       pltpu.VMEM_SHARED((Np, D), F32)],
    )
    def k(pay_hbm, idx_hbm, zero_hbm, out, idx_v, pay_v, table):
        w = _wid()
        cid = lax.axis_index("c")
        sid = lax.axis_index("s")
        sl = pl.ds(sid * rows_per_sub, rows_per_sub)
        pltpu.sync_copy(zero_hbm.at[sl], table.at[sl])
        plsc.subcore_barrier()
        tc = (nchunks - w + NW - 1) // NW

        def body(j, _):
            c = w + j * NW
            base = c * CH
            pltpu.sync_copy(idx_hbm.at[c], idx_v)
            pltpu.sync_copy(pay_hbm.at[pl.ds(base, CH)], pay_v)
            pltpu.sync_copy(pay_v, table.at[idx_v], add=True)
            return _

        lax.fori_loop(0, tc, body, None)
        plsc.subcore_barrier()
        pltpu.sync_copy(table.at[sl], out.at[cid, sl])

    return k


def _make_scatter2(E, Np, D):
    nchunks = E // CH
    rows_per_sub = Np // NS

    @functools.partial(
        pl.kernel,
        out_type=[jax.ShapeDtypeStruct((NC, Np, D), F32),
                  jax.ShapeDtypeStruct((NC, Np, D), F32)],
        mesh=_sc_mesh(),
        compiler_params=_SC_PARAMS,
        scratch_types=[pltpu.VMEM((CH,), jnp.int32),
                       pltpu.VMEM((CH, D), F32),
                       pltpu.VMEM_SHARED((Np, D), F32),
                       pltpu.VMEM_SHARED((Np, D), F32)],
    )
    def k(ps_hbm, pr_hbm, rs_hbm, rr_hbm, zero_hbm, out_s, out_r,
          idx_v, pay_v, tab_s, tab_r):
        w = _wid()
        cid = lax.axis_index("c")
        sid = lax.axis_index("s")
        sl = pl.ds(sid * rows_per_sub, rows_per_sub)
        pltpu.sync_copy(zero_hbm.at[sl], tab_s.at[sl])
        pltpu.sync_copy(zero_hbm.at[sl], tab_r.at[sl])
        plsc.subcore_barrier()
        tc = (nchunks - w + NW - 1) // NW

        def body(j, _):
            c = w + j * NW
            base = c * CH
            pltpu.sync_copy(rs_hbm.at[c], idx_v)
            pltpu.sync_copy(ps_hbm.at[pl.ds(base, CH)], pay_v)
            pltpu.sync_copy(pay_v, tab_s.at[idx_v], add=True)
            pltpu.sync_copy(rr_hbm.at[c], idx_v)
            pltpu.sync_copy(pr_hbm.at[pl.ds(base, CH)], pay_v)
            pltpu.sync_copy(pay_v, tab_r.at[idx_v], add=True)
            return _

        lax.fori_loop(0, tc, body, None)
        plsc.subcore_barrier()
        pltpu.sync_copy(tab_s.at[sl], out_s.at[cid, sl])
        pltpu.sync_copy(tab_r.at[sl], out_r.at[cid, sl])

    return k


def _edge_fwd_body(s_ref, r_ref, eW1_ref, eb1_ref, eW2_ref, eb2_ref,
                   nW1e_ref, m_ref, sg1_ref, sg2_ref, se_ref):
    S = s_ref[...]
    R = r_ref[...]
    dpos = S[:, 3:5] - R[:, 3:5]
    dpos = jnp.where(dpos > BOX / 2, dpos - BOX, dpos)
    dpos = jnp.where(dpos <= -BOX / 2, dpos + BOX, dpos)
    Ef = jnp.concatenate([S[:, 0:3], R[:, 0:3], dpos], axis=1)
    BF = jnp.bfloat16
    z1 = (_mmb(Ef, eW1_ref[...]) + eb1_ref[...]).astype(BF)
    A1, s1 = _spsg(z1)
    z2 = (_mmb(A1, eW2_ref[...]) + eb2_ref[...]).astype(BF)
    Eh, s2 = _spsg(z2)
    M = _mmb(Eh, nW1e_ref[...])
    m_ref[...] = jnp.concatenate(
        [M, jnp.zeros((M.shape[0], 12), F32)], axis=1)
    sg1_ref[...] = s1
    sg2_ref[...] = s2

    @pl.when(pl.program_id(0) == 0)
    def _():
        se_ref[...] = jnp.zeros_like(se_ref)

    ones = jnp.ones((1, Eh.shape[0]), BF)
    se_ref[...] += jnp.dot(ones, Eh, preferred_element_type=F32)


def _node_global_body(n_real, t_ref, agg_ref, se_ref,
                      nW1n_ref, nb1_ref, nW2_ref, nb2_ref, nW3_ref, nb3_ref,
                      gW1_ref, gb1_ref, gW2_ref, gb2_ref, lT_ref,
                      nW1nT_ref, nW2T_ref, nW3T_ref, gW1T_ref, gW2T_ref,
                      g_ref, dvnp_ref, dse_ref):
    Vnp = t_ref[...][:, 0:3]
    agg2 = agg_ref[...]
    agg = agg2[0, :, 0:100] + agg2[1, :, 0:100]
    zN1 = _mm(Vnp, nW1n_ref[...]) + agg + nb1_ref[...]
    Nh1, s1 = _spsg(zN1)
    zN2 = _mm(Nh1, nW2_ref[...]) + nb2_ref[...]
    Nh2, s2 = _spsg(zN2)
    zN3 = _mm(Nh2, nW3_ref[...]) + nb3_ref[...]
    Nh, s3 = _spsg(zN3)
    rows = lax.broadcasted_iota(jnp.int32, Nh.shape, 0)
    SN = jnp.sum(jnp.where(rows < n_real, Nh, 0.0), axis=0, keepdims=True)
    Gin = jnp.concatenate([se_ref[...], SN], axis=1)
    zG1 = _mm(Gin, gW1_ref[...]) + gb1_ref[...]
    Gh1, sG1 = _spsg(zG1)
    zG2 = _mm(Gh1, gW2_ref[...]) + gb2_ref[...]
    dGh1 = _mm(lT_ref[...] * _sg(zG2), gW2T_ref[...])
    dGin = _mm(dGh1 * sG1, gW1T_ref[...])
    dSE = dGin[:, 0:150]
    dSN = dGin[:, 150:250]
    dNh2 = _mm(dSN * s3, nW3T_ref[...])
    dNh1 = _mm(dNh2 * s2, nW2T_ref[...])
    G = dNh1 * s1
    dvnp = _mm(G, nW1nT_ref[...])
    g_ref[...] = jnp.concatenate(
        [G, jnp.zeros((G.shape[0], 12), F32)], axis=1)
    dvnp_ref[...] = jnp.concatenate(
        [dvnp, jnp.zeros((dvnp.shape[0], 13), F32)], axis=1)
    dse_ref[...] = dSE


def _edge_bwd_body(sg1_ref, sg2_ref, grr_ref, dse_ref,
                   nW1eT_ref, eW2T_ref, eW1T_ref,
                   ps_ref, pr_ref):
    dEh = dse_ref[...] + _mmb(grr_ref[...][:, 0:100], nW1eT_ref[...])
    dEh1 = _mmb(dEh * sg2_ref[...].astype(F32), eW2T_ref[...])
    dEf = _mmb(dEh1 * sg1_ref[...].astype(F32), eW1T_ref[...])
    zpad = jnp.zeros((dEf.shape[0], 11), F32)
    ps_ref[...] = jnp.concatenate(
        [dEf[:, 0:3], dEf[:, 6:8], zpad], axis=1)
    pr_ref[...] = jnp.concatenate(
        [dEf[:, 3:6], -dEf[:, 6:8], zpad], axis=1)


def _final_body(v_ref, dt_ref, ts_ref, tr_ref, dvnp_ref, out_ref):
    n = v_ref.shape[1]
    V = v_ref[0]
    ts2 = ts_ref[...]
    tr2 = tr_ref[...]
    ts = (ts2[0] + ts2[1])[:n]
    tr = (tr2[0] + tr2[1])[:n]
    dvnp_n = dvnp_ref[...][:n]
    dq = ts[:, 1:3] + tr[:, 1:3] + dvnp_n[:, 1:3]
    dvp = ts[:, 3:5] + tr[:, 3:5]
    dt = dt_ref[...]
    q = V[:, 1:3] + dt * dq
    q = jnp.where(q >= BOX / 2, q - BOX, q)
    q = jnp.where(q < -BOX / 2, q + BOX, q)
    p = V[:, 3:5] - dt * dvp
    out_ref[...] = jnp.concatenate([V[:, 0:1], q, p], axis=1)[None]


def _full(shape):
    return pl.BlockSpec(shape, lambda: tuple(0 for _ in shape))


def kernel(state, R_s, R_r, dt, eW1, eb1, eW2, eb2, nW1, nb1, nW2, nb2,
           nW3, nb3, gW1, gb1, gW2, gb2, lW, lb):
    B, N, _ = state.shape
    E = R_s.shape[1]
    Np = ((N + CH - 1) // CH) * CH
    EO = eW1.shape[1]
    NO = nW1.shape[1]
    BLK = 2000
    grid_e = E // BLK

    V = state[0]
    rs = R_s[0, :, 0].astype(jnp.int32)
    rr = R_r[0, :, 0].astype(jnp.int32)
    rs2 = rs.reshape(E // CH, CH)
    rr2 = rr.reshape(E // CH, CH)

    T16 = jnp.concatenate(
        [V[:, 0:1], V[:, 3:5], V[:, 1:3], jnp.zeros((N, 11), F32)], axis=1)
    T16 = jnp.concatenate([T16, jnp.zeros((Np - N, 16), F32)], axis=0)

    nW1n, nW1e = nW1[:3], nW1[3:]
    zeros112 = jnp.zeros((Np, 112), F32)
    zeros16 = jnp.zeros((Np, 16), F32)

    S16, R16 = _make_gather2(E, 16)(T16, rs2, rr2)

    BF = jnp.bfloat16
    M112, SG1, SG2, SE = pl.pallas_call(
        _edge_fwd_body,
        grid=(grid_e,),
        in_specs=[
            pl.BlockSpec((BLK, 16), lambda i: (i, 0)),
            pl.BlockSpec((BLK, 16), lambda i: (i, 0)),
            pl.BlockSpec((8, EO), lambda i: (0, 0)),
            pl.BlockSpec((1, EO), lambda i: (0, 0)),
            pl.BlockSpec((EO, EO), lambda i: (0, 0)),
            pl.BlockSpec((1, EO), lambda i: (0, 0)),
            pl.BlockSpec((EO, NO), lambda i: (0, 0)),
        ],
        out_specs=[
            pl.BlockSpec((BLK, 112), lambda i: (i, 0)),
            pl.BlockSpec((BLK, EO), lambda i: (i, 0)),
            pl.BlockSpec((BLK, EO), lambda i: (i, 0)),
            pl.BlockSpec((1, EO), lambda i: (0, 0)),
        ],
        out_shape=[
            jax.ShapeDtypeStruct((E, 112), F32),
            jax.ShapeDtypeStruct((E, EO), BF),
            jax.ShapeDtypeStruct((E, EO), BF),
            jax.ShapeDtypeStruct((1, EO), F32),
        ],
    )(S16, R16, eW1.astype(BF), eb1.reshape(1, EO), eW2.astype(BF),
      eb2.reshape(1, EO), nW1e.astype(BF))

    aggM = _make_scatter1(E, Np, 112)(M112, rr2, zeros112)

    G112, dVnp16, dSE = pl.pallas_call(
        functools.partial(_node_global_body, N),
        in_specs=[
            _full((Np, 16)), _full((NC, Np, 112)), _full((1, EO)),
            _full((3, NO)), _full((1, NO)),
            _full((NO, NO)), _full((1, NO)),
            _full((NO, NO)), _full((1, NO)),
            _full((EO + NO, NO)), _full((1, NO)),
            _full((NO, NO)), _full((1, NO)),
            _full((1, NO)),
            _full((NO, 3)), _full((NO, NO)), _full((NO, NO)),
            _full((NO, EO + NO)), _full((NO, NO)),
        ],
        out_specs=[_full((Np, 112)), _full((Np, 16)), _full((1, EO))],
        out_shape=[
            jax.ShapeDtypeStruct((Np, 112), F32),
            jax.ShapeDtypeStruct((Np, 16), F32),
            jax.ShapeDtypeStruct((1, EO), F32),
        ],
    )(T16, aggM, SE,
      nW1n, nb1.reshape(1, NO), nW2, nb2.reshape(1, NO), nW3,
      nb3.reshape(1, NO), gW1, gb1.reshape(1, NO), gW2, gb2.reshape(1, NO),
      lW.reshape(1, NO),
      nW1n.T, nW2.T, nW3.T, gW1.T, gW2.T)

    Grr = _make_gather1(E, 112)(G112, rr2)

    Prs, Prr = pl.pallas_call(
        _edge_bwd_body,
        grid=(grid_e,),
        in_specs=[
            pl.BlockSpec((BLK, EO), lambda i: (i, 0)),
            pl.BlockSpec((BLK, EO), lambda i: (i, 0)),
            pl.BlockSpec((BLK, 112), lambda i: (i, 0)),
            pl.BlockSpec((1, EO), lambda i: (0, 0)),
            pl.BlockSpec((NO, EO), lambda i: (0, 0)),
            pl.BlockSpec((EO, EO), lambda i: (0, 0)),
            pl.BlockSpec((EO, 8), lambda i: (0, 0)),
        ],
        out_specs=[
            pl.BlockSpec((BLK, 16), lambda i: (i, 0)),
            pl.BlockSpec((BLK, 16), lambda i: (i, 0)),
        ],
        out_shape=[
            jax.ShapeDtypeStruct((E, 16), F32),
            jax.ShapeDtypeStruct((E, 16), F32),
        ],
    )(SG1, SG2, Grr, dSE, nW1e.T.astype(BF), eW2.T.astype(BF),
      eW1.T.astype(BF))

    TS, TR = _make_scatter2(E, Np, 16)(Prs, Prr, rs2, rr2, zeros16)

    out = pl.pallas_call(
        _final_body,
        in_specs=[
            _full((B, N, 5)),
            _full((N, 1)),
            _full((NC, Np, 16)), _full((NC, Np, 16)),
            _full((Np, 16)),
        ],
        out_specs=_full((B, N, 5)),
        out_shape=jax.ShapeDtypeStruct((B, N, 5), F32),
    )(state, dt[0].reshape(N, 1), TS, TR, dVnp16)

    return out


# speedup vs baseline: 19.7156x; 1.2297x over previous
import functools

import jax
import jax.numpy as jnp
from jax import lax
from jax.experimental import pallas as pl
from jax.experimental.pallas import tpu as pltpu
from jax.experimental.pallas import tpu_sc as plsc

BOX = 6.0
NC, NS = 2, 16
NW = NC * NS
CH = 128
F32 = jnp.float32


def _sp(x):
    return jnp.maximum(x, 0.0) + jnp.log1p(jnp.exp(-jnp.abs(x)))


def _sg(x):
    return 1.0 / (1.0 + jnp.exp(-x))


def _spsg(x):
    t = jnp.exp(-jnp.abs(x))
    sp = jnp.maximum(x, 0.0) + jnp.log1p(t)
    sg = 1.0 - jnp.exp(-sp)
    return sp, sg


def _mm(a, b):
    return jnp.dot(a, b, preferred_element_type=F32)


def _mmb(a, b):
    return jnp.dot(a.astype(jnp.bfloat16), b, preferred_element_type=F32)


def _sc_mesh():
    return plsc.VectorSubcoreMesh(core_axis_name="c", subcore_axis_name="s")


_SC_PARAMS = pltpu.CompilerParams(use_tc_tiling_on_sc=False)


def _wid():
    return lax.axis_index("s") * NC + lax.axis_index("c")


def _chunk_layout(nchunks):
    base = nchunks // NW
    extra = nchunks % NW
    return base, extra


def _worker_start(w, base, extra):
    return w * base + jnp.minimum(w, extra)


def _make_gather2(E, D):
    nchunks = E // CH
    base_n, extra = _chunk_layout(nchunks)
    maxc = base_n + (1 if extra else 0)

    @functools.partial(
        pl.kernel,
        out_type=[jax.ShapeDtypeStruct((E, D), F32),
                  jax.ShapeDtypeStruct((E, D), F32)],
        mesh=_sc_mesh(),
        compiler_params=_SC_PARAMS,
        scratch_types=[pltpu.VMEM((maxc, CH), jnp.int32),
                       pltpu.VMEM((maxc, CH), jnp.int32),
                       pltpu.VMEM((CH, D), F32), pltpu.VMEM((CH, D), F32),
                       pltpu.VMEM((CH, D), F32), pltpu.VMEM((CH, D), F32),
                       pltpu.SemaphoreType.DMA, pltpu.SemaphoreType.DMA,
                       pltpu.SemaphoreType.DMA, pltpu.SemaphoreType.DMA],
    )
    def k(t_hbm, rs_hbm, rr_hbm, s_out, r_out,
          idxs, idxr, bs0, bs1, br0, br1, ss0, ss1, sr0, sr1):
        w = _wid()
        cw = _worker_start(w, base_n, extra)
        sbuf = (bs0, bs1)
        rbuf = (br0, br1)
        ssem = (ss0, ss1)
        rsem = (sr0, sr1)
        pltpu.sync_copy(rs_hbm.at[pl.ds(cw, base_n)],
                        idxs.at[pl.ds(0, base_n)])
        pltpu.sync_copy(rr_hbm.at[pl.ds(cw, base_n)],
                        idxr.at[pl.ds(0, base_n)])
        if extra:
            @pl.when(w < extra)
            def _():
                pltpu.sync_copy(rs_hbm.at[cw + base_n], idxs.at[base_n])
                pltpu.sync_copy(rr_hbm.at[cw + base_n], idxr.at[base_n])

        def start(j):
            b = j & 1
            pltpu.make_async_copy(t_hbm.at[idxs.at[j]], sbuf[b], ssem[b]).start()
            pltpu.make_async_copy(t_hbm.at[idxr.at[j]], rbuf[b], rsem[b]).start()

        def drain(j):
            b = j & 1
            pltpu.make_async_copy(t_hbm.at[idxs.at[j]], sbuf[b], ssem[b]).wait()
            pltpu.make_async_copy(t_hbm.at[idxr.at[j]], rbuf[b], rsem[b]).wait()
            off = (cw + j) * CH
            pltpu.sync_copy(sbuf[b], s_out.at[pl.ds(off, CH)])
            pltpu.sync_copy(rbuf[b], r_out.at[pl.ds(off, CH)])

        start(0)
        for j in range(1, base_n):
            start(j)
            drain(j - 1)
        if extra:
            @pl.when(w < extra)
            def _():
                start(base_n)
        drain(base_n - 1)
        if extra:
            @pl.when(w < extra)
            def _():
                drain(base_n)

    return k


def _make_gather1(E, D):
    nchunks = E // CH
    base_n, extra = _chunk_layout(nchunks)
    maxc = base_n + (1 if extra else 0)

    @functools.partial(
        pl.kernel,
        out_type=jax.ShapeDtypeStruct((E, D), F32),
        mesh=_sc_mesh(),
        compiler_params=_SC_PARAMS,
        scratch_types=[pltpu.VMEM((maxc, CH), jnp.int32),
                       pltpu.VMEM((CH, D), F32), pltpu.VMEM((CH, D), F32),
                       pltpu.SemaphoreType.DMA, pltpu.SemaphoreType.DMA],
    )
    def k(t_hbm, rr_hbm, out, idxr, b0, b1, s0, s1):
        w = _wid()
        cw = _worker_start(w, base_n, extra)
        buf = (b0, b1)
        sem = (s0, s1)
        pltpu.sync_copy(rr_hbm.at[pl.ds(cw, base_n)],
                        idxr.at[pl.ds(0, base_n)])
        if extra:
            @pl.when(w < extra)
            def _():
                pltpu.sync_copy(rr_hbm.at[cw + base_n], idxr.at[base_n])

        def start(j):
            b = j & 1
            pltpu.make_async_copy(t_hbm.at[idxr.at[j]], buf[b], sem[b]).start()

        def drain(j):
            b = j & 1
            pltpu.make_async_copy(t_hbm.at[idxr.at[j]], buf[b], sem[b]).wait()
            pltpu.sync_copy(buf[b], out.at[pl.ds((cw + j) * CH, CH)])

        start(0)
        for j in range(1, base_n):
            start(j)
            drain(j - 1)
        if extra:
            @pl.when(w < extra)
            def _():
                start(base_n)
        drain(base_n - 1)
        if extra:
            @pl.when(w < extra)
            def _():
                drain(base_n)

    return k


def _make_scatter1(E, Np, D):
    nchunks = E // CH
    base_n, extra = _chunk_layout(nchunks)
    maxc = base_n + (1 if extra else 0)
    rows_per_sub = Np // NS

    @functools.partial(
        pl.kernel,
        out_type=jax.ShapeDtypeStruct((NC, Np, D), F32),
        mesh=_sc_mesh(),
        compiler_params=_SC_PARAMS,
        scratch_types=[pltpu.VMEM((maxc, CH), jnp.int32),
                       pltpu.VMEM((CH, D), F32), pltpu.VMEM((CH, D), F32),
                       pltpu.VMEM_SHARED((Np, D), F32),
                       pltpu.SemaphoreType.DMA, pltpu.SemaphoreType.DMA],
    )
    def k(pay_hbm, idx_hbm, zero_hbm, out, idxv, p0, p1, table, s0, s1):
        w = _wid()
        cid = lax.axis_index("c")
        sid = lax.axis_index("s")
        sl = pl.ds(sid * rows_per_sub, rows_per_sub)
        pltpu.sync_copy(zero_hbm.at[sl], table.at[sl])
        cw = _worker_start(w, base_n, extra)
        buf = (p0, p1)
        sem = (s0, s1)
        pltpu.sync_copy(idx_hbm.at[pl.ds(cw, base_n)],
                        idxv.at[pl.ds(0, base_n)])
        if extra:
            @pl.when(w < extra)
            def _():
                pltpu.sync_copy(idx_hbm.at[cw + base_n], idxv.at[base_n])
        plsc.subcore_barrier()

        def start(j):
            b = j & 1
            pltpu.make_async_copy(
                pay_hbm.at[pl.ds((cw + j) * CH, CH)], buf[b], sem[b]).start()

        def drain(j):
            b = j & 1
            pltpu.make_async_copy(
                pay_hbm.at[pl.ds((cw + j) * CH, CH)], buf[b], sem[b]).wait()
            pltpu.sync_copy(buf[b], table.at[idxv.at[j]], add=True)

        start(0)
        for j in range(1, base_n):
            start(j)
            drain(j - 1)
        if extra:
            @pl.when(w < extra)
            def _():
                start(base_n)
        drain(base_n - 1)
        if extra:
            @pl.when(w < extra)
            def _():
                drain(base_n)
        plsc.subcore_barrier()
        pltpu.sync_copy(table.at[sl], out.at[cid, sl])

    return k


def _make_scatter2(E, Np, D):
    nchunks = E // CH
    base_n, extra = _chunk_layout(nchunks)
    maxc = base_n + (1 if extra else 0)
    rows_per_sub = Np // NS

    @functools.partial(
        pl.kernel,
        out_type=[jax.ShapeDtypeStruct((NC, Np, D), F32),
                  jax.ShapeDtypeStruct((NC, Np, D), F32)],
        mesh=_sc_mesh(),
        compiler_params=_SC_PARAMS,
        scratch_types=[pltpu.VMEM((maxc, CH), jnp.int32),
                       pltpu.VMEM((maxc, CH), jnp.int32),
                       pltpu.VMEM((CH, D), F32), pltpu.VMEM((CH, D), F32),
                       pltpu.VMEM((CH, D), F32), pltpu.VMEM((CH, D), F32),
                       pltpu.VMEM_SHARED((Np, D), F32),
                       pltpu.VMEM_SHARED((Np, D), F32),
                       pltpu.SemaphoreType.DMA, pltpu.SemaphoreType.DMA,
                       pltpu.SemaphoreType.DMA, pltpu.SemaphoreType.DMA],
    )
    def k(ps_hbm, pr_hbm, rs_hbm, rr_hbm, zero_hbm, out_s, out_r,
          idxs, idxr, ps0, ps1, pr0, pr1, tab_s, tab_r, ss0, ss1, sr0, sr1):
        w = _wid()
        cid = lax.axis_index("c")
        sid = lax.axis_index("s")
        sl = pl.ds(sid * rows_per_sub, rows_per_sub)
        pltpu.sync_copy(zero_hbm.at[sl], tab_s.at[sl])
        pltpu.sync_copy(zero_hbm.at[sl], tab_r.at[sl])
        cw = _worker_start(w, base_n, extra)
        sbuf = (ps0, ps1)
        rbuf = (pr0, pr1)
        ssem = (ss0, ss1)
        rsem = (sr0, sr1)
        pltpu.sync_copy(rs_hbm.at[pl.ds(cw, base_n)],
                        idxs.at[pl.ds(0, base_n)])
        pltpu.sync_copy(rr_hbm.at[pl.ds(cw, base_n)],
                        idxr.at[pl.ds(0, base_n)])
        if extra:
            @pl.when(w < extra)
            def _():
                pltpu.sync_copy(rs_hbm.at[cw + base_n], idxs.at[base_n])
                pltpu.sync_copy(rr_hbm.at[cw + base_n], idxr.at[base_n])
        plsc.subcore_barrier()

        def start(j):
            b = j & 1
            off = pl.ds((cw + j) * CH, CH)
            pltpu.make_async_copy(ps_hbm.at[off], sbuf[b], ssem[b]).start()
            pltpu.make_async_copy(pr_hbm.at[off], rbuf[b], rsem[b]).start()

        def drain(j):
            b = j & 1
            off = pl.ds((cw + j) * CH, CH)
            pltpu.make_async_copy(ps_hbm.at[off], sbuf[b], ssem[b]).wait()
            pltpu.make_async_copy(pr_hbm.at[off], rbuf[b], rsem[b]).wait()
            pltpu.sync_copy(sbuf[b], tab_s.at[idxs.at[j]], add=True)
            pltpu.sync_copy(rbuf[b], tab_r.at[idxr.at[j]], add=True)

        start(0)
        for j in range(1, base_n):
            start(j)
            drain(j - 1)
        if extra:
            @pl.when(w < extra)
            def _():
                start(base_n)
        drain(base_n - 1)
        if extra:
            @pl.when(w < extra)
            def _():
                drain(base_n)
        plsc.subcore_barrier()
        pltpu.sync_copy(tab_s.at[sl], out_s.at[cid, sl])
        pltpu.sync_copy(tab_r.at[sl], out_r.at[cid, sl])

    return k


def _edge_fwd_body(s_ref, r_ref, eW1_ref, eW2_ref,
                   nW1e_ref, m_ref, sg1_ref, sg2_ref, se_ref):
    S = s_ref[...]
    R = r_ref[...]
    dpos = S[:, 3:5] - R[:, 3:5]
    dpos = jnp.where(dpos > BOX / 2, dpos - BOX, dpos)
    dpos = jnp.where(dpos <= -BOX / 2, dpos + BOX, dpos)
    BF = jnp.bfloat16
    n = S.shape[0]
    ones = jnp.ones((n, 1), BF)
    Ef = jnp.concatenate(
        [S[:, 0:3], R[:, 0:3], dpos, jnp.ones((n, 1), F32)], axis=1)
    z1 = _mmb(Ef, eW1_ref[...]).astype(BF)
    A1, s1 = _spsg(z1)
    z2 = _mmb(jnp.concatenate([A1, ones], axis=1), eW2_ref[...]).astype(BF)
    Eh, s2 = _spsg(z2)
    m_ref[...] = _mmb(Eh, nW1e_ref[...])
    sg1_ref[...] = s1
    sg2_ref[...] = s2

    @pl.when(pl.program_id(0) == 0)
    def _():
        se_ref[...] = jnp.zeros_like(se_ref)

    se_ref[...] += jnp.dot(jnp.ones((1, n), BF), Eh,
                           preferred_element_type=F32)


def _node_global_body(n_real, t_ref, agg_ref, se_ref,
                      nW1n_ref, nb1_ref, nW2_ref, nb2_ref, nW3_ref, nb3_ref,
                      gW1_ref, gb1_ref, gW2_ref, gb2_ref, lT_ref,
                      nW1nT_ref, nW2T_ref, nW3T_ref, gW1T_ref, gW2T_ref,
                      g_ref, dvnp_ref, dse_ref):
    Vnp = t_ref[...][:, 0:3]
    agg2 = agg_ref[...]
    agg = agg2[0, :, 0:100] + agg2[1, :, 0:100]
    zN1 = _mm(Vnp, nW1n_ref[...]) + agg + nb1_ref[...]
    Nh1, s1 = _spsg(zN1)
    zN2 = _mm(Nh1, nW2_ref[...]) + nb2_ref[...]
    Nh2, s2 = _spsg(zN2)
    zN3 = _mm(Nh2, nW3_ref[...]) + nb3_ref[...]
    Nh, s3 = _spsg(zN3)
    rows = lax.broadcasted_iota(jnp.int32, Nh.shape, 0)
    SN = jnp.sum(jnp.where(rows < n_real, Nh, 0.0), axis=0, keepdims=True)
    Gin = jnp.concatenate([se_ref[...], SN], axis=1)
    zG1 = _mm(Gin, gW1_ref[...]) + gb1_ref[...]
    Gh1, sG1 = _spsg(zG1)
    zG2 = _mm(Gh1, gW2_ref[...]) + gb2_ref[...]
    dGh1 = _mm(lT_ref[...] * _sg(zG2), gW2T_ref[...])
    dGin = _mm(dGh1 * sG1, gW1T_ref[...])
    dSE = dGin[:, 0:150]
    dSN = dGin[:, 150:250]
    dNh2 = _mm(dSN * s3, nW3T_ref[...])
    dNh1 = _mm(dNh2 * s2, nW2T_ref[...])
    G = dNh1 * s1
    dvnp = _mm(G, nW1nT_ref[...])
    g_ref[...] = jnp.concatenate(
        [G, jnp.zeros((G.shape[0], 12), F32)], axis=1)
    dvnp_ref[...] = jnp.concatenate(
        [dvnp, jnp.zeros((dvnp.shape[0], 13), F32)], axis=1)
    dse_ref[...] = dSE


def _edge_bwd_body(sg1_ref, sg2_ref, grr_ref, dse_ref,
                   nW1eT_ref, eW2T_ref, eW1T_ref,
                   ps_ref, pr_ref):
    dEh = dse_ref[...] + _mmb(grr_ref[...][:, 0:100], nW1eT_ref[...])
    dEh1 = _mmb(dEh * sg2_ref[...].astype(F32), eW2T_ref[...])
    dEf = _mmb(dEh1 * sg1_ref[...].astype(F32), eW1T_ref[...])
    zpad = jnp.zeros((dEf.shape[0], 11), F32)
    ps_ref[...] = jnp.concatenate(
        [dEf[:, 0:3], dEf[:, 6:8], zpad], axis=1)
    pr_ref[...] = jnp.concatenate(
        [dEf[:, 3:6], -dEf[:, 6:8], zpad], axis=1)


def _final_body(v_ref, dt_ref, ts_ref, tr_ref, dvnp_ref, out_ref):
    n = v_ref.shape[1]
    V = v_ref[0]
    ts2 = ts_ref[...]
    tr2 = tr_ref[...]
    ts = (ts2[0] + ts2[1])[:n]
    tr = (tr2[0] + tr2[1])[:n]
    dvnp_n = dvnp_ref[...][:n]
    dq = ts[:, 1:3] + tr[:, 1:3] + dvnp_n[:, 1:3]
    dvp = ts[:, 3:5] + tr[:, 3:5]
    dt = dt_ref[...]
    q = V[:, 1:3] + dt * dq
    q = jnp.where(q >= BOX / 2, q - BOX, q)
    q = jnp.where(q < -BOX / 2, q + BOX, q)
    p = V[:, 3:5] - dt * dvp
    out_ref[...] = jnp.concatenate([V[:, 0:1], q, p], axis=1)[None]


def _full(shape):
    return pl.BlockSpec(shape, lambda: tuple(0 for _ in shape))


def kernel(state, R_s, R_r, dt, eW1, eb1, eW2, eb2, nW1, nb1, nW2, nb2,
           nW3, nb3, gW1, gb1, gW2, gb2, lW, lb):
    B, N, _ = state.shape
    E = R_s.shape[1]
    Np = ((N + CH - 1) // CH) * CH
    EO = eW1.shape[1]
    NO = nW1.shape[1]
    BLK = 2000
    grid_e = E // BLK

    V = state[0]
    rs = R_s[0, :, 0].astype(jnp.int32)
    rr = R_r[0, :, 0].astype(jnp.int32)
    rs2 = rs.reshape(E // CH, CH)
    rr2 = rr.reshape(E // CH, CH)

    T16 = jnp.concatenate(
        [V[:, 0:1], V[:, 3:5], V[:, 1:3], jnp.zeros((N, 11), F32)], axis=1)
    T16 = jnp.concatenate([T16, jnp.zeros((Np - N, 16), F32)], axis=0)

    nW1n, nW1e = nW1[:3], nW1[3:]
    zeros112 = jnp.zeros((Np, 112), F32)
    zeros16 = jnp.zeros((Np, 16), F32)

    S16, R16 = _make_gather2(E, 16)(T16, rs2, rr2)

    BF = jnp.bfloat16
    eW1a = jnp.concatenate([eW1, eb1[None]], axis=0).astype(BF)
    eW2a = jnp.concatenate([eW2, eb2[None]], axis=0).astype(BF)
    nW1e112 = jnp.concatenate(
        [nW1e, jnp.zeros((EO, 12), F32)], axis=1).astype(BF)
    M112, SG1, SG2, SE = pl.pallas_call(
        _edge_fwd_body,
        grid=(grid_e,),
        in_specs=[
            pl.BlockSpec((BLK, 16), lambda i: (i, 0)),
            pl.BlockSpec((BLK, 16), lambda i: (i, 0)),
            pl.BlockSpec((9, EO), lambda i: (0, 0)),
            pl.BlockSpec((EO + 1, EO), lambda i: (0, 0)),
            pl.BlockSpec((EO, 112), lambda i: (0, 0)),
        ],
        out_specs=[
            pl.BlockSpec((BLK, 112), lambda i: (i, 0)),
            pl.BlockSpec((BLK, EO), lambda i: (i, 0)),
            pl.BlockSpec((BLK, EO), lambda i: (i, 0)),
            pl.BlockSpec((1, EO), lambda i: (0, 0)),
        ],
        out_shape=[
            jax.ShapeDtypeStruct((E, 112), F32),
            jax.ShapeDtypeStruct((E, EO), BF),
            jax.ShapeDtypeStruct((E, EO), BF),
            jax.ShapeDtypeStruct((1, EO), F32),
        ],
    )(S16, R16, eW1a, eW2a, nW1e112)

    aggM = _make_scatter1(E, Np, 112)(M112, rr2, zeros112)

    G112, dVnp16, dSE = pl.pallas_call(
        functools.partial(_node_global_body, N),
        in_specs=[
            _full((Np, 16)), _full((NC, Np, 112)), _full((1, EO)),
            _full((3, NO)), _full((1, NO)),
            _full((NO, NO)), _full((1, NO)),
            _full((NO, NO)), _full((1, NO)),
            _full((EO + NO, NO)), _full((1, NO)),
            _full((NO, NO)), _full((1, NO)),
            _full((1, NO)),
            _full((NO, 3)), _full((NO, NO)), _full((NO, NO)),
            _full((NO, EO + NO)), _full((NO, NO)),
        ],
        out_specs=[_full((Np, 112)), _full((Np, 16)), _full((1, EO))],
        out_shape=[
            jax.ShapeDtypeStruct((Np, 112), F32),
            jax.ShapeDtypeStruct((Np, 16), F32),
            jax.ShapeDtypeStruct((1, EO), F32),
        ],
    )(T16, aggM, SE,
      nW1n, nb1.reshape(1, NO), nW2, nb2.reshape(1, NO), nW3,
      nb3.reshape(1, NO), gW1, gb1.reshape(1, NO), gW2, gb2.reshape(1, NO),
      lW.reshape(1, NO),
      nW1n.T, nW2.T, nW3.T, gW1.T, gW2.T)

    Grr = _make_gather1(E, 112)(G112, rr2)

    Prs, Prr = pl.pallas_call(
        _edge_bwd_body,
        grid=(grid_e,),
        in_specs=[
            pl.BlockSpec((BLK, EO), lambda i: (i, 0)),
            pl.BlockSpec((BLK, EO), lambda i: (i, 0)),
            pl.BlockSpec((BLK, 112), lambda i: (i, 0)),
            pl.BlockSpec((1, EO), lambda i: (0, 0)),
            pl.BlockSpec((NO, EO), lambda i: (0, 0)),
            pl.BlockSpec((EO, EO), lambda i: (0, 0)),
            pl.BlockSpec((EO, 8), lambda i: (0, 0)),
        ],
        out_specs=[
            pl.BlockSpec((BLK, 16), lambda i: (i, 0)),
            pl.BlockSpec((BLK, 16), lambda i: (i, 0)),
        ],
        out_shape=[
            jax.ShapeDtypeStruct((E, 16), F32),
            jax.ShapeDtypeStruct((E, 16), F32),
        ],
    )(SG1, SG2, Grr, dSE, nW1e.T.astype(BF), eW2.T.astype(BF),
      eW1.T.astype(BF))

    TS, TR = _make_scatter2(E, Np, 16)(Prs, Prr, rs2, rr2, zeros16)

    out = pl.pallas_call(
        _final_body,
        in_specs=[
            _full((B, N, 5)),
            _full((N, 1)),
            _full((NC, Np, 16)), _full((NC, Np, 16)),
            _full((Np, 16)),
        ],
        out_specs=_full((B, N, 5)),
        out_shape=jax.ShapeDtypeStruct((B, N, 5), F32),
    )(state, dt[0].reshape(N, 1), TS, TR, dVnp16)

    return out
